# baseline (speedup 1.0000x reference)
import functools

import jax
import jax.numpy as jnp
from jax import lax
from jax.experimental import pallas as pl
from jax.experimental.pallas import tpu as pltpu
from jax.experimental.pallas import tpu_sc as plsc

EMB_DIM = 32
B = 16384
MARGIN = 1.0
NC = 2
NS = 16
L = 16
NW = NC * NS
BW = B // NW
NGROUP = BW // L
DEPTH = 8
GW = EMB_DIM * L

EMAIN = 999936
EBLK = 7812
CD = [(d // 8) * (EBLK * 1024) + (d % 8) * 128 for d in range(EMB_DIM)]


def _rsqrt(a):
    i = plsc.bitcast(a, jnp.int32)
    i = jnp.int32(0x5F3759DF) - (i >> 1)
    y = plsc.bitcast(i, jnp.float32)
    for _ in range(3):
        y = y * (1.5 - 0.5 * a * y * y)
    return y


def _score(hh, tt, rr, hr, ht, rt):
    rh = _rsqrt(hh)
    rt_ = _rsqrt(tt)
    s2 = rr + 2.0 + 2.0 * (hr * rh - ht * (rh * rt_) - rt * rt_)
    s2 = jnp.maximum(s2, 0.0)
    return s2 * _rsqrt(s2 + 1e-30)


def _sc_body(idx_hbm, ent_hbm, tail_hbm, rel_hbm, out_hbm,
             idx_v, w0, w1, w2, w3, r0, r1, tmp_v, accv, esem, rsem, tsem):
    wid = lax.axis_index("s") * NC + lax.axis_index("c")
    pltpu.sync_copy(idx_hbm.at[wid], idx_v)
    wbufs = [w0, w1, w2, w3]
    rbufs = [r0, r1]

    def fire_group(g):
        slot = jnp.bitwise_and(g, DEPTH - 1)
        for k in range(4):
            e = jnp.minimum(idx_v[k, pl.ds(g * L, L)], EMAIN - 1)
            eb = ((e >> 7) << 10) + jnp.bitwise_and(e, 127)
            for d in range(EMB_DIM):
                pltpu.make_async_copy(
                    ent_hbm.at[eb + CD[d]],
                    wbufs[k].at[pl.ds(slot * GW + d * L, L)], esem).start()
        for k in range(2):
            pltpu.make_async_copy(
                rel_hbm.at[idx_v.at[4 + k, pl.ds(g * L, L)]],
                rbufs[k].at[pl.ds(slot * L, L)], rsem).start()

    def wait_group(g):
        slot = jnp.bitwise_and(g, DEPTH - 1)
        for k in range(4):
            pltpu.make_async_copy(
                ent_hbm.at[pl.ds(0, GW)],
                wbufs[k].at[pl.ds(slot * GW, GW)], esem).wait()
        for k in range(2):
            pltpu.make_async_copy(
                rel_hbm.at[pl.ds(0, L)],
                rbufs[k].at[pl.ds(slot * L, L)], rsem).wait()

    def fix_tail(g):
        slot = jnp.bitwise_and(g, DEPTH - 1)
        evs = [idx_v[k, pl.ds(g * L, L)] for k in range(4)]
        masks = [e >= EMAIN for e in evs]
        anym = masks[0] | masks[1] | masks[2] | masks[3]
        cnt = plsc.all_reduce_population_count(anym)

        @pl.when(cnt[0] > 0)
        def _():
            for k in range(4):
                et = jnp.clip(evs[k] - EMAIN, 0, 63)
                for d in range(EMB_DIM):
                    pltpu.make_async_copy(
                        tail_hbm.at[et * EMB_DIM + d],
                        tmp_v.at[pl.ds(d * L, L)], tsem).start()
                pltpu.make_async_copy(
                    tail_hbm.at[pl.ds(0, GW)], tmp_v, tsem).wait()
                for d in range(EMB_DIM):
                    sl = pl.ds(slot * GW + d * L, L)
                    main = wbufs[k][sl]
                    tv = tmp_v[pl.ds(d * L, L)]
                    wbufs[k][sl] = jnp.where(masks[k], tv, main)

    iota = lax.iota(jnp.int32, L)

    def compute_group(g, acc):
        slot = jnp.bitwise_and(g, DEPTH - 1)
        row = slot * L + iota
        z = jnp.zeros((L,), jnp.float32)
        p = [z] * 6
        n = [z] * 6
        for d in range(EMB_DIM):
            col = jnp.full((L,), d, jnp.int32)
            h = w0[pl.ds(slot * GW + d * L, L)]
            t = w1[pl.ds(slot * GW + d * L, L)]
            r = plsc.load_gather(r0, [row, col])
            p = [p[0] + h * h, p[1] + t * t, p[2] + r * r,
                 p[3] + h * r, p[4] + h * t, p[5] + r * t]
            h = w2[pl.ds(slot * GW + d * L, L)]
            t = w3[pl.ds(slot * GW + d * L, L)]
            r = plsc.load_gather(r1, [row, col])
            n = [n[0] + h * h, n[1] + t * t, n[2] + r * r,
                 n[3] + h * r, n[4] + h * t, n[5] + r * t]
        ps = _score(*p)
        ns = _score(*n)
        return acc + jnp.maximum(ps - ns + MARGIN, 0.0)

    for g in range(DEPTH - 1):
        fire_group(jnp.int32(g))

    def body(g, acc):
        @pl.when(g < NGROUP - (DEPTH - 1))
        def _():
            fire_group(g + (DEPTH - 1))
        wait_group(g)
        fix_tail(g)
        return compute_group(g, acc)

    acc = lax.fori_loop(0, NGROUP, body, jnp.zeros((L,), jnp.float32))
    accv[...] = acc
    pltpu.sync_copy(accv, out_hbm.at[wid])


_sc_call = functools.partial(
    pl.kernel,
    out_type=jax.ShapeDtypeStruct((NW, L), jnp.float32),
    mesh=plsc.VectorSubcoreMesh(core_axis_name="c", subcore_axis_name="s"),
    compiler_params=pltpu.CompilerParams(needs_layout_passes=False),
    scratch_types=[
        pltpu.VMEM((8, BW), jnp.int32),
        pltpu.VMEM((DEPTH * GW,), jnp.float32),
        pltpu.VMEM((DEPTH * GW,), jnp.float32),
        pltpu.VMEM((DEPTH * GW,), jnp.float32),
        pltpu.VMEM((DEPTH * GW,), jnp.float32),
        pltpu.VMEM((DEPTH * L, 4 * EMB_DIM), jnp.float32),
        pltpu.VMEM((DEPTH * L, 4 * EMB_DIM), jnp.float32),
        pltpu.VMEM((GW,), jnp.float32),
        pltpu.VMEM((L,), jnp.float32),
        pltpu.SemaphoreType.DMA,
        pltpu.SemaphoreType.DMA,
        pltpu.SemaphoreType.DMA,
    ],
)(_sc_body)


def kernel(pos_exmpls, neg_exmpls, ent_emb, rel_emb):
    ids = jnp.stack([pos_exmpls[:, 0], pos_exmpls[:, 2],
                     neg_exmpls[:, 0], neg_exmpls[:, 2],
                     pos_exmpls[:, 1], neg_exmpls[:, 1],
                     jnp.zeros((B,), jnp.int32), jnp.zeros((B,), jnp.int32)],
                    axis=0)
    idx = ids.reshape(8, NW, BW).transpose(1, 0, 2)
    ent_flat = (ent_emb[:EMAIN].reshape(EBLK, 128, 4, 8)
                .transpose(2, 0, 3, 1).reshape(-1))
    tail_flat = ent_emb[EMAIN:].reshape(-1)
    rel128 = jnp.pad(rel_emb, ((0, 0), (0, 128 - EMB_DIM)))
    partial = _sc_call(idx, ent_flat, tail_flat, rel128)
    return jnp.sum(partial) / jnp.float32(B)


# speedup vs baseline: 1.0362x; 1.0362x over previous
import functools

import jax
import jax.numpy as jnp
from jax import lax
from jax.experimental import pallas as pl
from jax.experimental.pallas import tpu as pltpu
from jax.experimental.pallas import tpu_sc as plsc

EMB_DIM = 32
B = 16384
MARGIN = 1.0
NC = 2
NS = 16
L = 16
NW = NC * NS
BW = B // NW
NGROUP = BW // L
DEPTH = 4
GW = EMB_DIM * L

EMAIN = 999936
EBLK = 7812
CD = [(d // 8) * (EBLK * 1024) + (d % 8) * 128 for d in range(EMB_DIM)]


def _rsqrt(a):
    i = plsc.bitcast(a, jnp.int32)
    i = jnp.int32(0x5F3759DF) - (i >> 1)
    y = plsc.bitcast(i, jnp.float32)
    for _ in range(3):
        y = y * (1.5 - 0.5 * a * y * y)
    return y


def _score(hh, tt, rr, hr, ht, rt):
    rh = _rsqrt(hh)
    rt_ = _rsqrt(tt)
    s2 = rr + 2.0 + 2.0 * (hr * rh - ht * (rh * rt_) - rt * rt_)
    s2 = jnp.maximum(s2, 0.0)
    return s2 * _rsqrt(s2 + 1e-30)


def _sc_body(idx_hbm, ent_hbm, tail_hbm, rel_hbm, out_hbm,
             idx_v, w0, w1, w2, w3, r0, r1, tmp_v, accv, esem, rsem, tsem):
    wid = lax.axis_index("s") * NC + lax.axis_index("c")
    pltpu.sync_copy(idx_hbm.at[wid], idx_v)
    wbufs = [w0, w1, w2, w3]
    rbufs = [r0, r1]

    def fire_group(g):
        slot = jnp.bitwise_and(g, DEPTH - 1)
        for k in range(4):
            e = jnp.minimum(idx_v[k, pl.ds(g * L, L)], EMAIN - 1)
            eb = ((e >> 7) << 10) + jnp.bitwise_and(e, 127)
            for d in range(EMB_DIM):
                pltpu.make_async_copy(
                    ent_hbm.at[eb + CD[d]],
                    wbufs[k].at[pl.ds(slot * GW + d * L, L)], esem).start()
        for k in range(2):
            pltpu.make_async_copy(
                rel_hbm.at[idx_v.at[4 + k, pl.ds(g * L, L)]],
                rbufs[k].at[pl.ds(slot * L, L)], rsem).start()

    def wait_group(g):
        slot = jnp.bitwise_and(g, DEPTH - 1)
        for k in range(4):
            pltpu.make_async_copy(
                ent_hbm.at[pl.ds(0, GW)],
                wbufs[k].at[pl.ds(slot * GW, GW)], esem).wait()
        for k in range(2):
            pltpu.make_async_copy(
                rel_hbm.at[pl.ds(0, L)],
                rbufs[k].at[pl.ds(slot * L, L)], rsem).wait()

    def fix_tail(g):
        slot = jnp.bitwise_and(g, DEPTH - 1)
        evs = [idx_v[k, pl.ds(g * L, L)] for k in range(4)]
        masks = [e >= EMAIN for e in evs]
        anym = masks[0] | masks[1] | masks[2] | masks[3]
        cnt = plsc.all_reduce_population_count(anym)

        @pl.when(cnt[0] > 0)
        def _():
            for k in range(4):
                et = jnp.clip(evs[k] - EMAIN, 0, 63)
                for d in range(EMB_DIM):
                    pltpu.make_async_copy(
                        tail_hbm.at[et * EMB_DIM + d],
                        tmp_v.at[pl.ds(d * L, L)], tsem).start()
                pltpu.make_async_copy(
                    tail_hbm.at[pl.ds(0, GW)], tmp_v, tsem).wait()
                for d in range(EMB_DIM):
                    sl = pl.ds(slot * GW + d * L, L)
                    main = wbufs[k][sl]
                    tv = tmp_v[pl.ds(d * L, L)]
                    wbufs[k][sl] = jnp.where(masks[k], tv, main)

    iota = lax.iota(jnp.int32, L)

    def compute_group(g, acc):
        slot = jnp.bitwise_and(g, DEPTH - 1)
        row = slot * L + iota
        z = jnp.zeros((L,), jnp.float32)
        p = [z] * 6
        n = [z] * 6
        for d in range(EMB_DIM):
            col = jnp.full((L,), d, jnp.int32)
            h = w0[pl.ds(slot * GW + d * L, L)]
            t = w1[pl.ds(slot * GW + d * L, L)]
            r = plsc.load_gather(r0, [row, col])
            p = [p[0] + h * h, p[1] + t * t, p[2] + r * r,
                 p[3] + h * r, p[4] + h * t, p[5] + r * t]
            h = w2[pl.ds(slot * GW + d * L, L)]
            t = w3[pl.ds(slot * GW + d * L, L)]
            r = plsc.load_gather(r1, [row, col])
            n = [n[0] + h * h, n[1] + t * t, n[2] + r * r,
                 n[3] + h * r, n[4] + h * t, n[5] + r * t]
        ps = _score(*p)
        ns = _score(*n)
        return acc + jnp.maximum(ps - ns + MARGIN, 0.0)

    for g in range(DEPTH - 1):
        fire_group(jnp.int32(g))

    def body(g, acc):
        @pl.when(g < NGROUP - (DEPTH - 1))
        def _():
            fire_group(g + (DEPTH - 1))
        wait_group(g)
        fix_tail(g)
        return compute_group(g, acc)

    acc = lax.fori_loop(0, NGROUP, body, jnp.zeros((L,), jnp.float32))
    accv[...] = acc
    pltpu.sync_copy(accv, out_hbm.at[wid])


_sc_call = functools.partial(
    pl.kernel,
    out_type=jax.ShapeDtypeStruct((NW, L), jnp.float32),
    mesh=plsc.VectorSubcoreMesh(core_axis_name="c", subcore_axis_name="s"),
    compiler_params=pltpu.CompilerParams(needs_layout_passes=False),
    scratch_types=[
        pltpu.VMEM((8, BW), jnp.int32),
        pltpu.VMEM((DEPTH * GW,), jnp.float32),
        pltpu.VMEM((DEPTH * GW,), jnp.float32),
        pltpu.VMEM((DEPTH * GW,), jnp.float32),
        pltpu.VMEM((DEPTH * GW,), jnp.float32),
        pltpu.VMEM((DEPTH * L, 4 * EMB_DIM), jnp.float32),
        pltpu.VMEM((DEPTH * L, 4 * EMB_DIM), jnp.float32),
        pltpu.VMEM((GW,), jnp.float32),
        pltpu.VMEM((L,), jnp.float32),
        pltpu.SemaphoreType.DMA,
        pltpu.SemaphoreType.DMA,
        pltpu.SemaphoreType.DMA,
    ],
)(_sc_body)


def kernel(pos_exmpls, neg_exmpls, ent_emb, rel_emb):
    ids = jnp.stack([pos_exmpls[:, 0], pos_exmpls[:, 2],
                     neg_exmpls[:, 0], neg_exmpls[:, 2],
                     pos_exmpls[:, 1], neg_exmpls[:, 1],
                     jnp.zeros((B,), jnp.int32), jnp.zeros((B,), jnp.int32)],
                    axis=0)
    idx = ids.reshape(8, NW, BW).transpose(1, 0, 2)
    ent_flat = (ent_emb[:EMAIN].reshape(EBLK, 128, 4, 8)
                .transpose(2, 0, 3, 1).reshape(-1))
    tail_flat = ent_emb[EMAIN:].reshape(-1)
    rel128 = jnp.pad(rel_emb, ((0, 0), (0, 128 - EMB_DIM)))
    partial = _sc_call(idx, ent_flat, tail_flat, rel128)
    return jnp.sum(partial) / jnp.float32(B)
